# baseline (speedup 1.0000x reference)
import functools

import jax
import jax.numpy as jnp
from jax import lax
from jax.experimental import pallas as pl
from jax.experimental.pallas import tpu as pltpu
from jax.experimental.pallas import tpu_sc as plsc

N_NODES = 100000
N_EDGES = 3200000
N_EDGE_F = 16
GLOBAL_F = 64
NUM_GRAPHS = 512
HIDDEN = 8

NC = 2
NS = 16
NW = NC * NS
BLK = 128
KB = 16
NB = N_EDGES // BLK


def _sc_body(src_hbm, attr_hbm, batch_hbm, sums_out, cnt_out,
             idx_v, seg_v, attr_v, cnt16, z2, acc_sh, gsem, ssem):
    cid = lax.axis_index("c")
    sid = lax.axis_index("s")
    wid = sid * NC + cid

    iota = lax.iota(jnp.int32, 16)
    ones = jnp.ones((16,), jnp.float32)
    zeros = jnp.zeros((16,), jnp.float32)

    def _zero(r, _):
        cnt16[pl.ds(r * 16, 16)] = zeros
        z2[r, :] = zeros
        return 0
    lax.fori_loop(0, NUM_GRAPHS, _zero, 0)

    @pl.when(sid == 0)
    def _():
        pltpu.sync_copy(z2, acc_sh)

    plsc.subcore_barrier()

    nsb = NB // 8
    s0 = (nsb * wid) // NW
    s1 = (nsb * (wid + 1)) // NW
    b0 = s0 * 8
    b1 = s1 * 8

    def process(blk0, kb):
        pltpu.sync_copy(src_hbm.at[pl.ds(blk0, kb)], idx_v.at[pl.ds(0, kb)])
        gd = [pltpu.async_copy(batch_hbm.at[idx_v.at[j]], seg_v.at[j], gsem)
              for j in range(kb)]
        pltpu.sync_copy(attr_hbm.at[pl.ds(blk0 * BLK, kb * BLK)],
                        attr_v.at[pl.ds(0, kb * BLK)])
        for d in gd:
            d.wait()
        sd = [pltpu.async_copy(attr_v.at[pl.ds(j * BLK, BLK)],
                               acc_sh.at[seg_v.at[j]], ssem, add=True)
              for j in range(kb)]
        for j in range(kb):
            for t in range(BLK // 16):
                s = seg_v[j, pl.ds(t * 16, 16)]
                plsc.addupdate_scatter(cnt16, [s * 16 + iota], ones)
        for d in sd:
            d.wait()

    nf = (b1 - b0) // KB

    def chunk_body(i, _):
        process(b0 + i * KB, KB)
        return 0
    lax.fori_loop(0, nf, chunk_body, 0)

    def tail_body(b, _):
        process(b, 8)
        return 0
    lax.fori_loop(0, (b1 - b0 - nf * KB) // 8,
                  lambda i, _: tail_body(b0 + nf * KB + i * 8, 0), 0)

    plsc.subcore_barrier()

    pltpu.sync_copy(cnt16, cnt_out.at[wid])

    @pl.when(sid == 0)
    def _():
        pltpu.sync_copy(acc_sh, sums_out.at[cid])


_sc_seg = functools.partial(
    pl.kernel,
    out_type=[
        jax.ShapeDtypeStruct((NC, NUM_GRAPHS, N_EDGE_F), jnp.float32),
        jax.ShapeDtypeStruct((NW, NUM_GRAPHS * N_EDGE_F), jnp.float32),
    ],
    mesh=plsc.VectorSubcoreMesh(core_axis_name="c", subcore_axis_name="s"),
    scratch_types=[
        pltpu.VMEM((KB, BLK), jnp.int32),
        pltpu.VMEM((KB, BLK), jnp.int32),
        pltpu.VMEM((KB * BLK, N_EDGE_F), jnp.float32),
        pltpu.VMEM((NUM_GRAPHS * N_EDGE_F,), jnp.float32),
        pltpu.VMEM((NUM_GRAPHS, N_EDGE_F), jnp.float32),
        pltpu.VMEM_SHARED((NUM_GRAPHS, N_EDGE_F), jnp.float32),
        pltpu.SemaphoreType.DMA,
        pltpu.SemaphoreType.DMA,
    ],
    compiler_params=pltpu.CompilerParams(needs_layout_passes=False,
                                         use_tc_tiling_on_sc=False),
)(_sc_body)


def _mlp_body(sums_ref, cnt_ref, u_ref, w1u_ref, w1m_ref, b1_ref, w2_ref,
              b2_ref, o_ref):
    sums = sums_ref[0] + sums_ref[1]
    counts = jnp.sum(cnt_ref[...].reshape(NW, NUM_GRAPHS, N_EDGE_F),
                     axis=(0, 2))
    mean = sums / jnp.maximum(counts, 1.0)[:, None]
    h = jnp.dot(u_ref[...], w1u_ref[...], preferred_element_type=jnp.float32)
    h = h + jnp.dot(mean, w1m_ref[...], preferred_element_type=jnp.float32)
    h = jnp.maximum(h + b1_ref[...], 0.0)
    o = jnp.dot(h, w2_ref[...], preferred_element_type=jnp.float32)
    o_ref[...] = o + b2_ref[...]


def _mlp(sums_p, cnt_p, u, w1u_t, w1m_t, b1, w2_t, b2):
    return pl.pallas_call(
        _mlp_body,
        out_shape=jax.ShapeDtypeStruct((NUM_GRAPHS, GLOBAL_F), jnp.float32),
    )(sums_p, cnt_p, u, w1u_t, w1m_t, b1, w2_t, b2)


def kernel(x, edge_index, edge_attr, u, batch, W1, b1, W2, b2):
    src = edge_index[0].astype(jnp.int32).reshape(NB, BLK)
    batch32 = batch.astype(jnp.int32)
    sums_p, cnt_p = _sc_seg(src, edge_attr, batch32)
    w1u_t = W1[:, :GLOBAL_F].T
    w1m_t = W1[:, GLOBAL_F:].T
    w2_t = W2.T
    return _mlp(sums_p, cnt_p, u, w1u_t, w1m_t,
                b1.reshape(1, HIDDEN), w2_t, b2.reshape(1, GLOBAL_F))


# speedup vs baseline: 1.8975x; 1.8975x over previous
import functools

import jax
import jax.numpy as jnp
from jax import lax
from jax.experimental import pallas as pl
from jax.experimental.pallas import tpu as pltpu
from jax.experimental.pallas import tpu_sc as plsc

N_NODES = 100000
N_EDGES = 3200000
N_EDGE_F = 16
GLOBAL_F = 64
NUM_GRAPHS = 512
HIDDEN = 8

NC = 2
NS = 16
NW = NC * NS
BLK = 128
KB = 16
NB = N_EDGES // BLK
ACC = NUM_GRAPHS * N_EDGE_F


def _sc_body(src_hbm, attr_hbm, batch_hbm, sums_out, cnt_out,
             idx_v, seg_v, attr_v, acc_t, cnt_t, gsem):
    cid = lax.axis_index("c")
    sid = lax.axis_index("s")
    wid = sid * NC + cid

    ones = jnp.ones((16,), jnp.float32)
    zeros = jnp.zeros((16,), jnp.float32)

    def _zero(r, _):
        acc_t[pl.ds(r * 16, 16)] = zeros
        return 0
    lax.fori_loop(0, ACC // 16, _zero, 0)

    def _zero_c(r, _):
        cnt_t[pl.ds(r * 16, 16)] = zeros
        return 0
    lax.fori_loop(0, NUM_GRAPHS // 16, _zero_c, 0)

    nsb = NB // 8
    s0 = (nsb * wid) // NW
    s1 = (nsb * (wid + 1)) // NW
    b0 = s0 * 8
    b1 = s1 * 8

    def process(blk0, kb):
        pltpu.sync_copy(src_hbm.at[pl.ds(blk0, kb)], idx_v.at[pl.ds(0, kb)])
        gd = [pltpu.async_copy(batch_hbm.at[idx_v.at[j]], seg_v.at[j], gsem)
              for j in range(kb)]
        pltpu.sync_copy(attr_hbm.at[:, pl.ds(blk0 * BLK, kb * BLK)],
                        attr_v.at[:, pl.ds(0, kb * BLK)])
        for d in gd:
            d.wait()
        for j in range(kb):
            for t in range(BLK // 16):
                s = seg_v[j, pl.ds(t * 16, 16)]
                plsc.addupdate_scatter(cnt_t, [s], ones)
                e0 = j * BLK + t * 16
                for f in range(N_EDGE_F):
                    v = attr_v[f, pl.ds(e0, 16)]
                    plsc.addupdate_scatter(acc_t, [s + f * NUM_GRAPHS], v)

    nf = (b1 - b0) // KB

    def chunk_body(i, _):
        process(b0 + i * KB, KB)
        return 0
    lax.fori_loop(0, nf, chunk_body, 0)

    def tail_body(i, _):
        process(b0 + nf * KB + i * 8, 8)
        return 0
    lax.fori_loop(0, (b1 - b0 - nf * KB) // 8, tail_body, 0)

    pltpu.sync_copy(acc_t, sums_out.at[pl.ds(wid * ACC, ACC)])
    pltpu.sync_copy(cnt_t, cnt_out.at[pl.ds(wid * NUM_GRAPHS, NUM_GRAPHS)])


_sc_seg = functools.partial(
    pl.kernel,
    out_type=[
        jax.ShapeDtypeStruct((NW * ACC,), jnp.float32),
        jax.ShapeDtypeStruct((NW * NUM_GRAPHS,), jnp.float32),
    ],
    mesh=plsc.VectorSubcoreMesh(core_axis_name="c", subcore_axis_name="s"),
    scratch_types=[
        pltpu.VMEM((KB, BLK), jnp.int32),
        pltpu.VMEM((KB, BLK), jnp.int32),
        pltpu.VMEM((N_EDGE_F, KB * BLK), jnp.float32),
        pltpu.VMEM((ACC,), jnp.float32),
        pltpu.VMEM((NUM_GRAPHS,), jnp.float32),
        pltpu.SemaphoreType.DMA,
    ],
    compiler_params=pltpu.CompilerParams(needs_layout_passes=False,
                                         use_tc_tiling_on_sc=True),
)(_sc_body)


def _mlp_body(sums_ref, cnt_ref, u_ref, w1u_ref, w1m_ref, b1_ref, w2_ref,
              b2_ref, o_ref):
    sums_t = jnp.sum(sums_ref[...], axis=0)
    counts = jnp.sum(cnt_ref[...], axis=0)
    mean = (sums_t / jnp.maximum(counts, 1.0)[None, :]).T
    h = jnp.dot(u_ref[...], w1u_ref[...], preferred_element_type=jnp.float32)
    h = h + jnp.dot(mean, w1m_ref[...], preferred_element_type=jnp.float32)
    h = jnp.maximum(h + b1_ref[...], 0.0)
    o = jnp.dot(h, w2_ref[...], preferred_element_type=jnp.float32)
    o_ref[...] = o + b2_ref[...]


def _mlp(sums_p, cnt_p, u, w1u_t, w1m_t, b1, w2_t, b2):
    return pl.pallas_call(
        _mlp_body,
        out_shape=jax.ShapeDtypeStruct((NUM_GRAPHS, GLOBAL_F), jnp.float32),
    )(sums_p, cnt_p, u, w1u_t, w1m_t, b1, w2_t, b2)


def kernel(x, edge_index, edge_attr, u, batch, W1, b1, W2, b2):
    src = edge_index[0].astype(jnp.int32).reshape(NB, BLK)
    attr_t = edge_attr.T
    batch32 = batch.astype(jnp.int32)
    sums_p, cnt_p = _sc_seg(src, attr_t, batch32)
    w1u_t = W1[:, :GLOBAL_F].T
    w1m_t = W1[:, GLOBAL_F:].T
    w2_t = W2.T
    return _mlp(sums_p.reshape(NW, N_EDGE_F, NUM_GRAPHS),
                cnt_p.reshape(NW, NUM_GRAPHS),
                u, w1u_t, w1m_t,
                b1.reshape(1, HIDDEN), w2_t, b2.reshape(1, GLOBAL_F))


# speedup vs baseline: 2.2716x; 1.1971x over previous
import functools

import jax
import jax.numpy as jnp
from jax import lax
from jax.experimental import pallas as pl
from jax.experimental.pallas import tpu as pltpu
from jax.experimental.pallas import tpu_sc as plsc

N_NODES = 100000
N_EDGES = 3200000
N_EDGE_F = 16
GLOBAL_F = 64
NUM_GRAPHS = 512
HIDDEN = 8

NC = 2
NS = 16
NW = NC * NS
BLK = 128
KB = 16
NB = N_EDGES // BLK
ACC = NUM_GRAPHS * N_EDGE_F


def _sc_body(src_hbm, attr_hbm, batch_hbm, sums_out, cnt_out,
             idx_v, seg_v, attr_v, acc_t, cnt_t, gsem):
    cid = lax.axis_index("c")
    sid = lax.axis_index("s")
    wid = sid * NC + cid

    ones = jnp.ones((16,), jnp.float32)
    zeros = jnp.zeros((16,), jnp.float32)

    def _zero(r, _):
        acc_t[pl.ds(r * 16, 16)] = zeros
        return 0
    lax.fori_loop(0, ACC // 16, _zero, 0)

    def _zero_c(r, _):
        cnt_t[pl.ds(r * 16, 16)] = zeros
        return 0
    lax.fori_loop(0, NUM_GRAPHS // 16, _zero_c, 0)

    nsb = NB // 8
    s0 = (nsb * wid) // NW
    s1 = (nsb * (wid + 1)) // NW
    b0 = s0 * 8
    b1 = s1 * 8

    def process(blk0, kb):
        pltpu.sync_copy(src_hbm.at[pl.ds(blk0, kb)], idx_v.at[pl.ds(0, kb)])
        gd = [pltpu.async_copy(batch_hbm.at[idx_v.at[j]], seg_v.at[j], gsem)
              for j in range(kb)]
        pltpu.sync_copy(attr_hbm.at[:, pl.ds(blk0 * BLK, kb * BLK)],
                        attr_v.at[:, pl.ds(0, kb * BLK)])
        for d in gd:
            d.wait()
        for j in range(kb):
            for t in range(BLK // 16):
                s = seg_v[j, pl.ds(t * 16, 16)]
                e0 = j * BLK + t * 16
                vals = [attr_v[f, pl.ds(e0, 16)] for f in range(N_EDGE_F)]
                plsc.addupdate_scatter(cnt_t, [s], ones)
                for f in range(N_EDGE_F):
                    plsc.addupdate_scatter(acc_t, [s + f * NUM_GRAPHS],
                                           vals[f])

    nf = (b1 - b0) // KB

    def chunk_body(i, _):
        process(b0 + i * KB, KB)
        return 0
    lax.fori_loop(0, nf, chunk_body, 0)

    def tail_body(i, _):
        process(b0 + nf * KB + i * 8, 8)
        return 0
    lax.fori_loop(0, (b1 - b0 - nf * KB) // 8, tail_body, 0)

    pltpu.sync_copy(acc_t, sums_out.at[pl.ds(wid * ACC, ACC)])
    pltpu.sync_copy(cnt_t, cnt_out.at[pl.ds(wid * NUM_GRAPHS, NUM_GRAPHS)])


_sc_seg = functools.partial(
    pl.kernel,
    out_type=[
        jax.ShapeDtypeStruct((NW * ACC,), jnp.float32),
        jax.ShapeDtypeStruct((NW * NUM_GRAPHS,), jnp.float32),
    ],
    mesh=plsc.VectorSubcoreMesh(core_axis_name="c", subcore_axis_name="s"),
    scratch_types=[
        pltpu.VMEM((KB, BLK), jnp.int32),
        pltpu.VMEM((KB, BLK), jnp.int32),
        pltpu.VMEM((N_EDGE_F, KB * BLK), jnp.float32),
        pltpu.VMEM((ACC,), jnp.float32),
        pltpu.VMEM((NUM_GRAPHS,), jnp.float32),
        pltpu.SemaphoreType.DMA,
    ],
    compiler_params=pltpu.CompilerParams(needs_layout_passes=False,
                                         use_tc_tiling_on_sc=True),
)(_sc_body)


def _mlp_body(sums_ref, cnt_ref, u_ref, w1u_ref, w1m_ref, b1_ref, w2_ref,
              b2_ref, o_ref):
    sums_t = jnp.sum(sums_ref[...], axis=0)
    counts = jnp.sum(cnt_ref[...], axis=0)
    mean = (sums_t / jnp.maximum(counts, 1.0)[None, :]).T
    h = jnp.dot(u_ref[...], w1u_ref[...], preferred_element_type=jnp.float32)
    h = h + jnp.dot(mean, w1m_ref[...], preferred_element_type=jnp.float32)
    h = jnp.maximum(h + b1_ref[...], 0.0)
    o = jnp.dot(h, w2_ref[...], preferred_element_type=jnp.float32)
    o_ref[...] = o + b2_ref[...]


def _mlp(sums_p, cnt_p, u, w1u_t, w1m_t, b1, w2_t, b2):
    return pl.pallas_call(
        _mlp_body,
        out_shape=jax.ShapeDtypeStruct((NUM_GRAPHS, GLOBAL_F), jnp.float32),
    )(sums_p, cnt_p, u, w1u_t, w1m_t, b1, w2_t, b2)


def kernel(x, edge_index, edge_attr, u, batch, W1, b1, W2, b2):
    src = edge_index[0].astype(jnp.int32).reshape(NB, BLK)
    attr_t = edge_attr.T
    batch32 = batch.astype(jnp.int32)
    sums_p, cnt_p = _sc_seg(src, attr_t, batch32)
    w1u_t = W1[:, :GLOBAL_F].T
    w1m_t = W1[:, GLOBAL_F:].T
    w2_t = W2.T
    return _mlp(sums_p.reshape(NW, N_EDGE_F, NUM_GRAPHS),
                cnt_p.reshape(NW, NUM_GRAPHS),
                u, w1u_t, w1m_t,
                b1.reshape(1, HIDDEN), w2_t, b2.reshape(1, GLOBAL_F))


# speedup vs baseline: 3.0403x; 1.3384x over previous
import functools

import jax
import jax.numpy as jnp
from jax import lax
from jax.experimental import pallas as pl
from jax.experimental.pallas import tpu as pltpu
from jax.experimental.pallas import tpu_sc as plsc

N_NODES = 100000
N_EDGES = 3200000
N_EDGE_F = 16
GLOBAL_F = 64
NUM_GRAPHS = 512
HIDDEN = 8

NC = 2
NS = 16
NW = NC * NS
BLK = 128
C = 1024
NB = N_EDGES // BLK
ACC = NUM_GRAPHS * N_EDGE_F


def _sc_body(src_hbm, attr_hbm, batchw_hbm, sums_out, cnt_out,
             idx_v, attr_v, batch_w, acc_t, cnt_t, isem, asem):
    cid = lax.axis_index("c")
    sid = lax.axis_index("s")
    wid = sid * NC + cid

    ones = jnp.ones((16,), jnp.float32)
    zeros = jnp.zeros((16,), jnp.float32)

    bd = pltpu.async_copy(batchw_hbm, batch_w, asem)

    def _zero(r, _):
        acc_t[pl.ds(r * 16, 16)] = zeros
        return 0
    lax.fori_loop(0, ACC // 16, _zero, 0)

    def _zero_c(r, _):
        cnt_t[pl.ds(r * 16, 16)] = zeros
        return 0
    lax.fori_loop(0, NUM_GRAPHS // 16, _zero_c, 0)

    nsb = NB // 8
    s0 = (nsb * wid) // NW
    s1 = (nsb * (wid + 1)) // NW
    n = s1 - s0

    def start(i, par):
        blk0 = (s0 + i) * 8
        pltpu.async_copy(src_hbm.at[pl.ds(blk0, 8)], idx_v.at[par], isem)
        pltpu.async_copy(attr_hbm.at[:, pl.ds(blk0 * BLK, C)],
                         attr_v.at[:, pl.ds(par * C, C)], asem)

    def drain(par):
        pltpu.make_async_copy(src_hbm.at[pl.ds(0, 8)],
                              idx_v.at[par], isem).wait()
        pltpu.make_async_copy(attr_hbm.at[:, pl.ds(0, C)],
                              attr_v.at[:, pl.ds(par * C, C)], asem).wait()

    bd.wait()
    start(0, 0)

    def chunk_body(i, _):
        par = lax.rem(i, 2)
        drain(par)

        @pl.when(i + 1 < n)
        def _():
            start(i + 1, 1 - par)

        for g in range(C // 16):
            idx = idx_v[par, g // 8, pl.ds((g % 8) * 16, 16)]
            w = plsc.load_gather(batch_w, [idx >> 1])
            seg = (w >> ((idx & 1) << 4)) & 0xFFFF
            e0 = par * C + g * 16
            vals = [attr_v[f, pl.ds(e0, 16)] for f in range(N_EDGE_F)]
            plsc.addupdate_scatter(cnt_t, [seg], ones)
            for f in range(N_EDGE_F):
                plsc.addupdate_scatter(acc_t, [seg + f * NUM_GRAPHS],
                                       vals[f])
        return 0
    lax.fori_loop(0, n, chunk_body, 0)

    pltpu.sync_copy(acc_t, sums_out.at[pl.ds(wid * ACC, ACC)])
    pltpu.sync_copy(cnt_t, cnt_out.at[pl.ds(wid * NUM_GRAPHS, NUM_GRAPHS)])


_sc_seg = functools.partial(
    pl.kernel,
    out_type=[
        jax.ShapeDtypeStruct((NW * ACC,), jnp.float32),
        jax.ShapeDtypeStruct((NW * NUM_GRAPHS,), jnp.float32),
    ],
    mesh=plsc.VectorSubcoreMesh(core_axis_name="c", subcore_axis_name="s"),
    scratch_types=[
        pltpu.VMEM((2, 8, BLK), jnp.int32),
        pltpu.VMEM((N_EDGE_F, 2 * C), jnp.float32),
        pltpu.VMEM((N_NODES // 2,), jnp.int32),
        pltpu.VMEM((ACC,), jnp.float32),
        pltpu.VMEM((NUM_GRAPHS,), jnp.float32),
        pltpu.SemaphoreType.DMA,
        pltpu.SemaphoreType.DMA,
    ],
    compiler_params=pltpu.CompilerParams(needs_layout_passes=False,
                                         use_tc_tiling_on_sc=True),
)(_sc_body)


def _mlp_body(sums_ref, cnt_ref, u_ref, w1u_ref, w1m_ref, b1_ref, w2_ref,
              b2_ref, o_ref):
    sums_t = jnp.sum(sums_ref[...], axis=0)
    counts = jnp.sum(cnt_ref[...], axis=0)
    mean = (sums_t / jnp.maximum(counts, 1.0)[None, :]).T
    h = jnp.dot(u_ref[...], w1u_ref[...], preferred_element_type=jnp.float32)
    h = h + jnp.dot(mean, w1m_ref[...], preferred_element_type=jnp.float32)
    h = jnp.maximum(h + b1_ref[...], 0.0)
    o = jnp.dot(h, w2_ref[...], preferred_element_type=jnp.float32)
    o_ref[...] = o + b2_ref[...]


def _mlp(sums_p, cnt_p, u, w1u_t, w1m_t, b1, w2_t, b2):
    return pl.pallas_call(
        _mlp_body,
        out_shape=jax.ShapeDtypeStruct((NUM_GRAPHS, GLOBAL_F), jnp.float32),
    )(sums_p, cnt_p, u, w1u_t, w1m_t, b1, w2_t, b2)


def kernel(x, edge_index, edge_attr, u, batch, W1, b1, W2, b2):
    src = edge_index[0].astype(jnp.int32).reshape(NB, BLK)
    attr_t = edge_attr.T
    batch_w = jax.lax.bitcast_convert_type(
        batch.astype(jnp.int16).reshape(N_NODES // 2, 2), jnp.int32)
    sums_p, cnt_p = _sc_seg(src, attr_t, batch_w)
    w1u_t = W1[:, :GLOBAL_F].T
    w1m_t = W1[:, GLOBAL_F:].T
    w2_t = W2.T
    return _mlp(sums_p.reshape(NW, N_EDGE_F, NUM_GRAPHS),
                cnt_p.reshape(NW, NUM_GRAPHS),
                u, w1u_t, w1m_t,
                b1.reshape(1, HIDDEN), w2_t, b2.reshape(1, GLOBAL_F))


# speedup vs baseline: 3.2043x; 1.0539x over previous
import functools

import jax
import jax.numpy as jnp
from jax import lax
from jax.experimental import pallas as pl
from jax.experimental.pallas import tpu as pltpu
from jax.experimental.pallas import tpu_sc as plsc

N_NODES = 100000
N_EDGES = 3200000
N_EDGE_F = 16
GLOBAL_F = 64
NUM_GRAPHS = 512
HIDDEN = 8

NC = 2
NS = 16
NW = NC * NS
BLK = 128
C = 1024
NB = N_EDGES // BLK
ACC = NUM_GRAPHS * N_EDGE_F


def _sc_body(src_hbm, attr_hbm, batchw_hbm, sums_out, cnt_out,
             idx_v, attr_v, batch_w, acc_t, cnt_t, isem, asem):
    cid = lax.axis_index("c")
    sid = lax.axis_index("s")
    wid = sid * NC + cid

    ones = jnp.ones((16,), jnp.float32)
    zeros = jnp.zeros((16,), jnp.float32)

    bd = pltpu.async_copy(batchw_hbm, batch_w, asem)

    def _zero(r, _):
        acc_t[pl.ds(r * 16, 16)] = zeros
        return 0
    lax.fori_loop(0, ACC // 16, _zero, 0)

    def _zero_c(r, _):
        cnt_t[pl.ds(r * 16, 16)] = zeros
        return 0
    lax.fori_loop(0, NUM_GRAPHS // 16, _zero_c, 0)

    nsb = NB // 8
    s0 = (nsb * wid) // NW
    s1 = (nsb * (wid + 1)) // NW
    n = s1 - s0

    def start(i, par):
        blk0 = (s0 + i) * 8
        pltpu.async_copy(src_hbm.at[pl.ds(blk0, 8)], idx_v.at[par], isem)
        pltpu.async_copy(attr_hbm.at[:, pl.ds(blk0 * BLK, C)],
                         attr_v.at[:, pl.ds(par * C, C)], asem)

    def drain(par):
        pltpu.make_async_copy(src_hbm.at[pl.ds(0, 8)],
                              idx_v.at[par], isem).wait()
        pltpu.make_async_copy(attr_hbm.at[:, pl.ds(0, C)],
                              attr_v.at[:, pl.ds(par * C, C)], asem).wait()

    bd.wait()
    start(0, 0)

    def chunk_body(i, _):
        par = lax.rem(i, 2)
        drain(par)

        @pl.when(i + 1 < n)
        def _():
            start(i + 1, 1 - par)

        for g in range(C // 16):
            idx = idx_v[par, g // 8, pl.ds((g % 8) * 16, 16)]
            w = plsc.load_gather(batch_w, [idx >> 1])
            seg = (w >> ((idx & 1) << 4)) & 0xFFFF
            e0 = par * C + g * 16
            vals = [attr_v[f, pl.ds(e0, 16)] for f in range(N_EDGE_F)]
            plsc.addupdate_scatter(cnt_t, [seg], ones)
            for f in range(N_EDGE_F):
                plsc.addupdate_scatter(acc_t, [seg + f * NUM_GRAPHS],
                                       vals[f])
        return 0
    lax.fori_loop(0, n, chunk_body, 0)

    pltpu.sync_copy(acc_t, sums_out.at[pl.ds(wid * ACC, ACC)])
    pltpu.sync_copy(cnt_t, cnt_out.at[pl.ds(wid * NUM_GRAPHS, NUM_GRAPHS)])


_sc_seg = functools.partial(
    pl.kernel,
    out_type=[
        jax.ShapeDtypeStruct((NW * ACC,), jnp.float32),
        jax.ShapeDtypeStruct((NW * NUM_GRAPHS,), jnp.float32),
    ],
    mesh=plsc.VectorSubcoreMesh(core_axis_name="c", subcore_axis_name="s"),
    scratch_types=[
        pltpu.VMEM((2, 8, BLK), jnp.int32),
        pltpu.VMEM((N_EDGE_F, 2 * C), jnp.float32),
        pltpu.VMEM((N_NODES // 2,), jnp.int32),
        pltpu.VMEM((ACC,), jnp.float32),
        pltpu.VMEM((NUM_GRAPHS,), jnp.float32),
        pltpu.SemaphoreType.DMA,
        pltpu.SemaphoreType.DMA,
    ],
    compiler_params=pltpu.CompilerParams(needs_layout_passes=False,
                                         use_tc_tiling_on_sc=True),
)(_sc_body)


def _mlp_body(sums_ref, cnt_ref, u_ref, w1u_ref, w1m_ref, b1_ref, w2_ref,
              b2_ref, o_ref):
    sums_t = jnp.sum(sums_ref[...], axis=0)
    counts = jnp.sum(cnt_ref[...], axis=0)
    mean = (sums_t / jnp.maximum(counts, 1.0)[None, :]).T
    h = jnp.dot(u_ref[...], w1u_ref[...], preferred_element_type=jnp.float32)
    h = h + jnp.dot(mean, w1m_ref[...], preferred_element_type=jnp.float32)
    h = jnp.maximum(h + b1_ref[...], 0.0)
    o = jnp.dot(h, w2_ref[...], preferred_element_type=jnp.float32)
    o_ref[...] = o + b2_ref[...]


def _mlp(sums_p, cnt_p, u, w1u_t, w1m_t, b1, w2_t, b2):
    return pl.pallas_call(
        _mlp_body,
        out_shape=jax.ShapeDtypeStruct((NUM_GRAPHS, GLOBAL_F), jnp.float32),
    )(sums_p, cnt_p, u, w1u_t, w1m_t, b1, w2_t, b2)


def kernel(x, edge_index, edge_attr, u, batch, W1, b1, W2, b2):
    src = edge_index[0].astype(jnp.int32).reshape(NB, BLK)
    attr_t = edge_attr.T
    b32 = batch.astype(jnp.int32)
    batch_w = b32[0::2] | (b32[1::2] << 16)
    sums_p, cnt_p = _sc_seg(src, attr_t, batch_w)
    w1u_t = W1[:, :GLOBAL_F].T
    w1m_t = W1[:, GLOBAL_F:].T
    w2_t = W2.T
    return _mlp(sums_p.reshape(NW, N_EDGE_F, NUM_GRAPHS),
                cnt_p.reshape(NW, NUM_GRAPHS),
                u, w1u_t, w1m_t,
                b1.reshape(1, HIDDEN), w2_t, b2.reshape(1, GLOBAL_F))


# speedup vs baseline: 3.2685x; 1.0200x over previous
import functools

import jax
import jax.numpy as jnp
from jax import lax
from jax.experimental import pallas as pl
from jax.experimental.pallas import tpu as pltpu
from jax.experimental.pallas import tpu_sc as plsc

N_NODES = 100000
N_EDGES = 3200000
N_EDGE_F = 16
GLOBAL_F = 64
NUM_GRAPHS = 512
HIDDEN = 8

NC = 2
NS = 16
NW = NC * NS
BLK = 128
C = 1024
NB = N_EDGES // BLK
SPREAD = 4
ACC0 = NUM_GRAPHS * N_EDGE_F
ACC = ACC0 * SPREAD
CNTW = NUM_GRAPHS * SPREAD


def _sc_body(src_hbm, attr_hbm, batchw_hbm, sums_out, cnt_out,
             idx_v, attr_v, batch_w, acc_t, cnt_t, isem, asem):
    cid = lax.axis_index("c")
    sid = lax.axis_index("s")
    wid = sid * NC + cid

    ones = jnp.ones((16,), jnp.float32)
    zeros = jnp.zeros((16,), jnp.float32)
    lane4 = lax.iota(jnp.int32, 16) & 3

    bd = pltpu.async_copy(batchw_hbm, batch_w, asem)

    def _zero(r, _):
        acc_t[pl.ds(r * 16, 16)] = zeros
        return 0
    lax.fori_loop(0, ACC // 16, _zero, 0)

    def _zero_c(r, _):
        cnt_t[pl.ds(r * 16, 16)] = zeros
        return 0
    lax.fori_loop(0, CNTW // 16, _zero_c, 0)

    nsb = NB // 8
    s0 = (nsb * wid) // NW
    s1 = (nsb * (wid + 1)) // NW
    n = s1 - s0

    def start(i, par):
        blk0 = (s0 + i) * 8
        pltpu.async_copy(src_hbm.at[pl.ds(blk0, 8)], idx_v.at[par], isem)
        pltpu.async_copy(attr_hbm.at[:, pl.ds(blk0 * BLK, C)],
                         attr_v.at[:, pl.ds(par * C, C)], asem)

    def drain(par):
        pltpu.make_async_copy(src_hbm.at[pl.ds(0, 8)],
                              idx_v.at[par], isem).wait()
        pltpu.make_async_copy(attr_hbm.at[:, pl.ds(0, C)],
                              attr_v.at[:, pl.ds(par * C, C)], asem).wait()

    bd.wait()
    start(0, 0)

    def chunk_body(i, _):
        par = lax.rem(i, 2)
        drain(par)

        @pl.when(i + 1 < n)
        def _():
            start(i + 1, 1 - par)

        for g in range(C // 16):
            idx = idx_v[par, g // 8, pl.ds((g % 8) * 16, 16)]
            w = plsc.load_gather(batch_w, [idx >> 1])
            seg = (w >> ((idx & 1) << 4)) & 0xFFFF
            base = (seg << 2) | lane4
            e0 = par * C + g * 16
            vals = [attr_v[f, pl.ds(e0, 16)] for f in range(N_EDGE_F)]
            plsc.addupdate_scatter(cnt_t, [base], ones)
            for f in range(N_EDGE_F):
                plsc.addupdate_scatter(acc_t, [base + f * (4 * NUM_GRAPHS)],
                                       vals[f])
        return 0
    lax.fori_loop(0, n, chunk_body, 0)

    iota4 = lax.iota(jnp.int32, 16) * SPREAD

    def _fold(ref, nout):
        def body(r, _):
            b = r * 16 * SPREAD + iota4
            v = plsc.load_gather(ref, [b])
            for k in range(1, SPREAD):
                v = v + plsc.load_gather(ref, [b + k])
            ref[pl.ds(r * 16, 16)] = v
            return 0
        lax.fori_loop(0, nout // 16, body, 0)

    _fold(acc_t, ACC0)
    _fold(cnt_t, NUM_GRAPHS)
    pltpu.sync_copy(acc_t.at[pl.ds(0, ACC0)],
                    sums_out.at[pl.ds(wid * ACC0, ACC0)])
    pltpu.sync_copy(cnt_t.at[pl.ds(0, NUM_GRAPHS)],
                    cnt_out.at[pl.ds(wid * NUM_GRAPHS, NUM_GRAPHS)])


_sc_seg = functools.partial(
    pl.kernel,
    out_type=[
        jax.ShapeDtypeStruct((NW * ACC0,), jnp.float32),
        jax.ShapeDtypeStruct((NW * NUM_GRAPHS,), jnp.float32),
    ],
    mesh=plsc.VectorSubcoreMesh(core_axis_name="c", subcore_axis_name="s"),
    scratch_types=[
        pltpu.VMEM((2, 8, BLK), jnp.int32),
        pltpu.VMEM((N_EDGE_F, 2 * C), jnp.float32),
        pltpu.VMEM((N_NODES // 2,), jnp.int32),
        pltpu.VMEM((ACC,), jnp.float32),
        pltpu.VMEM((CNTW,), jnp.float32),
        pltpu.SemaphoreType.DMA,
        pltpu.SemaphoreType.DMA,
    ],
    compiler_params=pltpu.CompilerParams(needs_layout_passes=False,
                                         use_tc_tiling_on_sc=True),
)(_sc_body)


def _mlp_body(sums_ref, cnt_ref, u_ref, w1u_ref, w1m_ref, b1_ref, w2_ref,
              b2_ref, o_ref):
    sums_t = jnp.sum(sums_ref[...], axis=0)
    counts = jnp.sum(cnt_ref[...], axis=0)
    mean = (sums_t / jnp.maximum(counts, 1.0)[None, :]).T
    h = jnp.dot(u_ref[...], w1u_ref[...], preferred_element_type=jnp.float32)
    h = h + jnp.dot(mean, w1m_ref[...], preferred_element_type=jnp.float32)
    h = jnp.maximum(h + b1_ref[...], 0.0)
    o = jnp.dot(h, w2_ref[...], preferred_element_type=jnp.float32)
    o_ref[...] = o + b2_ref[...]


def _mlp(sums_p, cnt_p, u, w1u_t, w1m_t, b1, w2_t, b2):
    return pl.pallas_call(
        _mlp_body,
        out_shape=jax.ShapeDtypeStruct((NUM_GRAPHS, GLOBAL_F), jnp.float32),
    )(sums_p, cnt_p, u, w1u_t, w1m_t, b1, w2_t, b2)


def kernel(x, edge_index, edge_attr, u, batch, W1, b1, W2, b2):
    src = edge_index[0].astype(jnp.int32).reshape(NB, BLK)
    attr_t = edge_attr.T
    b32 = batch.astype(jnp.int32)
    batch_w = b32[0::2] | (b32[1::2] << 16)
    sums_p, cnt_p = _sc_seg(src, attr_t, batch_w)
    w1u_t = W1[:, :GLOBAL_F].T
    w1m_t = W1[:, GLOBAL_F:].T
    w2_t = W2.T
    return _mlp(sums_p.reshape(NW, N_EDGE_F, NUM_GRAPHS),
                cnt_p.reshape(NW, NUM_GRAPHS),
                u, w1u_t, w1m_t,
                b1.reshape(1, HIDDEN), w2_t, b2.reshape(1, GLOBAL_F))
